# Initial kernel scaffold; baseline (speedup 1.0000x reference)
import jax
import jax.numpy as jnp
from jax.experimental import pallas as pl


def kernel(x, pos_table):
    raise NotImplementedError("write your pallas kernel here")


# speedup vs baseline: 1.3986x; 1.3986x over previous
import jax
import jax.numpy as jnp
from jax.experimental import pallas as pl
from jax.experimental.pallas import tpu as pltpu

BATCH, MAXLEN, EMBED = 4, 2048, 1024
SEQ_BLK = 256


def _add_body(x_ref, pos_ref, o_ref):
    o_ref[...] = x_ref[...] + pos_ref[...]


def kernel(x, pos_table):
    grid = (BATCH, MAXLEN // SEQ_BLK)
    return pl.pallas_call(
        _add_body,
        grid=grid,
        in_specs=[
            pl.BlockSpec((1, SEQ_BLK, EMBED), lambda b, s: (b, s, 0)),
            pl.BlockSpec((SEQ_BLK, EMBED), lambda b, s: (s, 0)),
        ],
        out_specs=pl.BlockSpec((1, SEQ_BLK, EMBED), lambda b, s: (b, s, 0)),
        out_shape=jax.ShapeDtypeStruct((BATCH, MAXLEN, EMBED), jnp.float32),
    )(x, pos_table)


# speedup vs baseline: 1.4779x; 1.0567x over previous
import jax
import jax.numpy as jnp
from jax.experimental import pallas as pl
from jax.experimental.pallas import tpu as pltpu

BATCH, MAXLEN, EMBED = 4, 2048, 1024
SEQ_BLK = 256


def _add_body(x_ref, pos_ref, o_ref):
    o_ref[...] = x_ref[...] + pos_ref[...]


def kernel(x, pos_table):
    grid = (MAXLEN // SEQ_BLK, BATCH)
    return pl.pallas_call(
        _add_body,
        grid=grid,
        in_specs=[
            pl.BlockSpec((1, SEQ_BLK, EMBED), lambda s, b: (b, s, 0)),
            pl.BlockSpec((SEQ_BLK, EMBED), lambda s, b: (s, 0)),
        ],
        out_specs=pl.BlockSpec((1, SEQ_BLK, EMBED), lambda s, b: (b, s, 0)),
        out_shape=jax.ShapeDtypeStruct((BATCH, MAXLEN, EMBED), jnp.float32),
    )(x, pos_table)


# speedup vs baseline: 2.1542x; 1.4576x over previous
import jax
import jax.numpy as jnp
from jax.experimental import pallas as pl
from jax.experimental.pallas import tpu as pltpu

BATCH, MAXLEN, EMBED = 4, 2048, 1024
SEQ_BLK = 256


def _add_body(x_ref, pos_ref, o_ref):
    o_ref[...] = x_ref[...] + pos_ref[...][None, :, :]


def kernel(x, pos_table):
    grid = (MAXLEN // SEQ_BLK,)
    return pl.pallas_call(
        _add_body,
        grid=grid,
        in_specs=[
            pl.BlockSpec((BATCH, SEQ_BLK, EMBED), lambda s: (0, s, 0)),
            pl.BlockSpec((SEQ_BLK, EMBED), lambda s: (s, 0)),
        ],
        out_specs=pl.BlockSpec((BATCH, SEQ_BLK, EMBED), lambda s: (0, s, 0)),
        out_shape=jax.ShapeDtypeStruct((BATCH, MAXLEN, EMBED), jnp.float32),
    )(x, pos_table)


# speedup vs baseline: 2.1600x; 1.0027x over previous
import jax
import jax.numpy as jnp
from jax.experimental import pallas as pl
from jax.experimental.pallas import tpu as pltpu

BATCH, MAXLEN, EMBED = 4, 2048, 1024
SEQ_BLK = 512


def _add_body(x_ref, pos_ref, o_ref):
    o_ref[...] = x_ref[...] + pos_ref[...][None, :, :]


def kernel(x, pos_table):
    grid = (MAXLEN // SEQ_BLK,)
    return pl.pallas_call(
        _add_body,
        grid=grid,
        in_specs=[
            pl.BlockSpec((BATCH, SEQ_BLK, EMBED), lambda s: (0, s, 0)),
            pl.BlockSpec((SEQ_BLK, EMBED), lambda s: (s, 0)),
        ],
        out_specs=pl.BlockSpec((BATCH, SEQ_BLK, EMBED), lambda s: (0, s, 0)),
        out_shape=jax.ShapeDtypeStruct((BATCH, MAXLEN, EMBED), jnp.float32),
    )(x, pos_table)
